# Initial kernel scaffold; baseline (speedup 1.0000x reference)
import jax
import jax.numpy as jnp
from jax.experimental import pallas as pl


def kernel(z, codebook_weight):
    raise NotImplementedError("write your pallas kernel here")


# speedup vs baseline: 1.7219x; 1.7219x over previous
import functools

import jax
import jax.numpy as jnp
from jax.experimental import pallas as pl
from jax.experimental.pallas import tpu as pltpu

_B, _D, _H, _W = 32, 256, 32, 32
_HW = _H * _W
_K = 1024
_BETA = 0.2


def _vq_kernel(z_ref, cb_ref, q_ref, loss_ref):
    z_b = z_ref[0]
    cb = cb_ref[...]

    zsq = jnp.sum(z_b * z_b, axis=0, keepdims=True)
    cbsq = jnp.sum(cb * cb, axis=1, keepdims=True)
    m = jax.lax.dot_general(
        cb, z_b, (((1,), (0,)), ((), ())),
        preferred_element_type=jnp.float32,
        precision=jax.lax.Precision.DEFAULT,
    )
    dist = (zsq + cbsq) - 2.0 * m

    minval = jnp.min(dist, axis=0, keepdims=True)
    iota_k = jax.lax.broadcasted_iota(jnp.int32, (_K, _HW), 0)
    masked = jnp.where(dist == minval, iota_k, _K)
    idx = jnp.min(masked, axis=0, keepdims=True)
    onehot = (iota_k == idx).astype(jnp.float32)

    q = jax.lax.dot_general(
        cb, onehot, (((0,), (0,)), ((), ())),
        preferred_element_type=jnp.float32,
        precision=jax.lax.Precision.HIGHEST,
    )
    q_ref[0] = q

    r = z_b - q
    loss_ref[0, 0, 0] = jnp.sum(r * r)


@functools.partial(jax.jit, static_argnames=())
def kernel(z, codebook_weight):
    b, d, h, w = z.shape
    z3 = z.reshape(b, d, h * w)
    q3, loss_parts = pl.pallas_call(
        _vq_kernel,
        grid=(b,),
        in_specs=[
            pl.BlockSpec((1, d, h * w), lambda i: (i, 0, 0)),
            pl.BlockSpec((_K, d), lambda i: (0, 0)),
        ],
        out_specs=[
            pl.BlockSpec((1, d, h * w), lambda i: (i, 0, 0)),
            pl.BlockSpec((1, 1, 1), lambda i: (i, 0, 0), memory_space=pltpu.SMEM),
        ],
        out_shape=[
            jax.ShapeDtypeStruct((b, d, h * w), jnp.float32),
            jax.ShapeDtypeStruct((b, 1, 1), jnp.float32),
        ],
        compiler_params=pltpu.CompilerParams(
            dimension_semantics=("parallel",),
        ),
    )(z3, codebook_weight)
    quantized = q3.reshape(b, d, h, w)
    total = jnp.sum(loss_parts)
    codebook_loss = total / (b * h * w * d)
    commitment_loss = _BETA * codebook_loss
    return (quantized, codebook_loss, commitment_loss)


# speedup vs baseline: 2.7861x; 1.6180x over previous
import functools

import jax
import jax.numpy as jnp
from jax.experimental import pallas as pl
from jax.experimental.pallas import tpu as pltpu

_B, _D, _H, _W = 32, 256, 32, 32
_HW = _H * _W
_K = 1024
_BETA = 0.2


def _vq_kernel(z_ref, cb_ref, q_ref, loss_ref):
    z_b = z_ref[0]
    cb = cb_ref[...]

    zsq = jnp.sum(z_b * z_b, axis=0, keepdims=True)
    cbsq = jnp.sum(cb * cb, axis=1, keepdims=True)
    m = jax.lax.dot_general(
        cb, z_b, (((1,), (0,)), ((), ())),
        preferred_element_type=jnp.float32,
        precision=jax.lax.Precision.DEFAULT,
    )
    dist = (zsq + cbsq) - 2.0 * m

    minval = jnp.min(dist, axis=0, keepdims=True)
    iota_k = jax.lax.broadcasted_iota(jnp.int32, (_K, _HW), 0)
    masked = jnp.where(dist == minval, iota_k, _K)
    idx = jnp.min(masked, axis=0, keepdims=True)
    onehot = (iota_k == idx).astype(jnp.float32)

    q = jax.lax.dot_general(
        cb, onehot, (((0,), (0,)), ((), ())),
        preferred_element_type=jnp.float32,
        precision=jax.lax.Precision.DEFAULT,
    )
    q_ref[0] = q

    r = z_b - q
    loss_ref[0, 0, 0] = jnp.sum(r * r)


@functools.partial(jax.jit, static_argnames=())
def kernel(z, codebook_weight):
    b, d, h, w = z.shape
    z3 = z.reshape(b, d, h * w)
    q3, loss_parts = pl.pallas_call(
        _vq_kernel,
        grid=(b,),
        in_specs=[
            pl.BlockSpec((1, d, h * w), lambda i: (i, 0, 0)),
            pl.BlockSpec((_K, d), lambda i: (0, 0)),
        ],
        out_specs=[
            pl.BlockSpec((1, d, h * w), lambda i: (i, 0, 0)),
            pl.BlockSpec((1, 1, 1), lambda i: (i, 0, 0), memory_space=pltpu.SMEM),
        ],
        out_shape=[
            jax.ShapeDtypeStruct((b, d, h * w), jnp.float32),
            jax.ShapeDtypeStruct((b, 1, 1), jnp.float32),
        ],
        compiler_params=pltpu.CompilerParams(
            dimension_semantics=("parallel",),
        ),
    )(z3, codebook_weight)
    quantized = q3.reshape(b, d, h, w)
    total = jnp.sum(loss_parts)
    codebook_loss = total / (b * h * w * d)
    commitment_loss = _BETA * codebook_loss
    return (quantized, codebook_loss, commitment_loss)
